# baseline (speedup 1.0000x reference)
import functools

import jax
import jax.numpy as jnp
from jax import lax
from jax.experimental import pallas as pl
from jax.experimental.pallas import tpu as pltpu
from jax.experimental.pallas import tpu_sc as plsc

NC = 2
NS = 16
NW = NC * NS
LANES = 16
NBUF = 4

_SC_PARAMS = dict(compiler_params=pltpu.CompilerParams(
    use_tc_tiling_on_sc=False, needs_layout_passes=False))


@functools.lru_cache(maxsize=None)
def _build_gather(B, L, V, D):
    Nb = B // NW
    Lc = 2
    n_ch = L // Lc
    assert B % NW == 0 and L % (Lc * NBUF) == 0 and D == 2 * LANES
    assert Nb % LANES == 0

    mesh = plsc.VectorSubcoreMesh(core_axis_name="c", subcore_axis_name="s")

    @functools.partial(
        pl.kernel, mesh=mesh, **_SC_PARAMS,
        out_type=jax.ShapeDtypeStruct((L, D // 8, B // 128, 8, 128),
                                      jnp.float32),
        scratch_types=(
            [pltpu.VMEM((L, Nb), jnp.int32)]
            + [pltpu.VMEM((Nb, D), jnp.float32) for _ in range(Lc * NBUF)]
            + [pltpu.VMEM((D // 8, 8, Nb + 1), jnp.float32)
               for _ in range(Lc * NBUF)]
            + [pltpu.VMEM((L, D), jnp.float32)]
            + [pltpu.SemaphoreType.DMA for _ in range(2 * NBUF)]
        ),
    )
    def gk(trows, xT, pos_hbm, out, *refs):
        nslot = Lc * NBUF
        idx_all = refs[0]
        gbuf = refs[1:1 + nslot]
        tbuf = refs[1 + nslot:1 + 2 * nslot]
        pos_v = refs[1 + 2 * nslot]
        gsem = refs[2 + 2 * nslot:2 + 2 * nslot + NBUF]
        osem = refs[2 + 2 * nslot + NBUF:2 + 2 * nslot + 2 * NBUF]

        w = lax.axis_index("s") * NC + lax.axis_index("c")
        b0 = w * Nb
        pltpu.sync_copy(xT.at[:, pl.ds(b0, Nb)], idx_all)
        pltpu.sync_copy(pos_hbm, pos_v)
        dlo = lax.iota(jnp.int32, LANES)
        dhi = dlo + LANES
        dh_lo, dl_lo = dlo // 8, dlo % 8
        dh_hi, dl_hi = dhi // 8, dhi % 8

        def start_gathers(c, b):
            l0 = c * Lc
            for li in range(Lc):
                pltpu.make_async_copy(
                    trows.at[idx_all.at[l0 + li]], gbuf[Lc * b + li],
                    gsem[b]).start()

        def wait_gathers(c, b):
            l0 = c * Lc
            for li in range(Lc):
                pltpu.make_async_copy(
                    trows.at[idx_all.at[l0 + li]], gbuf[Lc * b + li],
                    gsem[b]).wait()

        def start_outs(c, b):
            l0 = c * Lc
            for li in range(Lc):
                pltpu.make_async_copy(
                    tbuf[Lc * b + li].at[:, :, pl.ds(0, Nb)],
                    out.at[l0 + li, :, w, :, :],
                    osem[b]).start()

        def wait_outs(c, b):
            l0 = c * Lc
            for li in range(Lc):
                pltpu.make_async_copy(
                    tbuf[Lc * b + li].at[:, :, pl.ds(0, Nb)],
                    out.at[l0 + li, :, w, :, :],
                    osem[b]).wait()

        start_gathers(0, 0)
        start_gathers(1, 1)

        def quad(pi, _):
            for b in range(NBUF):
                c = NBUF * pi + b
                wait_gathers(c, b)
                for li in range(Lc):
                    l = c * Lc + li
                    p0 = pos_v[l, pl.ds(0, LANES)]
                    p1 = pos_v[l, pl.ds(LANES, LANES)]
                    g2 = gbuf[Lc * b + li]
                    t2 = tbuf[Lc * b + li]

                    def jloop(j, _):
                        v0 = g2[j, pl.ds(0, LANES)] + p0
                        v1 = g2[j, pl.ds(LANES, LANES)] + p1
                        jsp = lax.broadcast(j, (LANES,)).astype(jnp.int32)
                        plsc.store_scatter(t2, [dh_lo, dl_lo, jsp], v0)
                        plsc.store_scatter(t2, [dh_hi, dl_hi, jsp], v1)
                        return 0

                    lax.fori_loop(0, Nb, jloop, 0, unroll=8)
                start_outs(c, b)
                b2 = (b + 2) % NBUF

                @pl.when(c >= 2)
                def _():
                    wait_outs(c - 2, b2)

                @pl.when(c + 2 < n_ch)
                def _():
                    start_gathers(c + 2, b2)
            return 0

        lax.fori_loop(0, n_ch // NBUF, quad, 0)
        for c in (n_ch - 2, n_ch - 1):
            wait_outs(c, c % NBUF)

    return gk


def kernel(x, token_table, pos_table):
    B, L = x.shape
    V, D = token_table.shape
    gk = _build_gather(B, L, V, D)
    x_T = jnp.swapaxes(x, 0, 1)
    out5 = gk(token_table, x_T, pos_table)
    return jnp.transpose(out5, (2, 4, 0, 1, 3)).reshape(B, L, D)
